# baseline (speedup 1.0000x reference)
import functools

import jax
import jax.numpy as jnp
from jax.experimental import pallas as pl
from jax.experimental.pallas import tpu as pltpu

_TOPK = 10
_PAD = 16
_INT_MAX = jnp.iinfo(jnp.int32).max


def _retrieval_kernel(nblk, bk, q_ref, k2_ref, keys_ref, vals_ref, idx_ref,
                      q2_ref, m_ref, s_ref, topv_ref, topi_ref, y_ref):
    j = pl.program_id(1)
    qn = q_ref.shape[0]

    @pl.when(j == 0)
    def _init():
        q = q_ref[:]
        q2_ref[:] = jnp.sum(q * q, axis=1, keepdims=True)
        m_ref[:] = jnp.full(m_ref.shape, -jnp.inf, jnp.float32)
        s_ref[:] = jnp.zeros(s_ref.shape, jnp.float32)
        topv_ref[:] = jnp.full(topv_ref.shape, -jnp.inf, jnp.float32)
        topi_ref[:] = jnp.full(topi_ref.shape, _INT_MAX, jnp.int32)

    kb = keys_ref[:]
    xy = jax.lax.dot_general(q_ref[:], kb, (((1,), (1,)), ((), ())),
                             preferred_element_type=jnp.float32)
    x = 2.0 * xy - (q2_ref[:] + k2_ref[0])

    theta = topv_ref[:, _TOPK - 1:_TOPK]
    cmp = x > theta
    counts = jnp.sum(cmp, axis=1, keepdims=True)
    trip = jnp.minimum(jnp.max(counts), _TOPK)
    y_ref[:] = jnp.where(cmp, x, -jnp.inf)

    lane = jax.lax.broadcasted_iota(jnp.int32, (qn, _PAD), 1)

    def _extract(t, m_acc):
        del t
        ids = j * bk + jax.lax.broadcasted_iota(jnp.int32, y_ref.shape, 1)
        y = y_ref[:]
        mx = jnp.max(y, axis=1, keepdims=True)
        hit = y == mx
        ii = jnp.min(jnp.where(hit, ids, _INT_MAX), axis=1, keepdims=True)
        y_ref[:] = jnp.where(ids == ii, -jnp.inf, y)
        cv = topv_ref[:]
        ci = topi_ref[:]
        beat = (cv > mx) | ((cv == mx) & (ci < ii))
        pos = jnp.sum(beat.astype(jnp.int32), axis=1, keepdims=True)
        sv = pltpu.roll(cv, 1, 1)
        si = pltpu.roll(ci, 1, 1)
        topv_ref[:] = jnp.where(lane < pos, cv, jnp.where(lane == pos, mx, sv))
        topi_ref[:] = jnp.where(lane < pos, ci, jnp.where(lane == pos, ii, si))
        return jnp.maximum(m_acc, mx)

    m_old = m_ref[:]
    m_new = jax.lax.fori_loop(0, trip, _extract, m_old)
    s_ref[:] = s_ref[:] * jnp.exp(m_old - m_new) + jnp.sum(
        jnp.exp(x - m_new), axis=1, keepdims=True)
    m_ref[:] = m_new

    @pl.when(j == nblk - 1)
    def _fin():
        vals_ref[:] = (topv_ref[:] - m_ref[:]) - jnp.log(s_ref[:])
        idx_ref[:] = topi_ref[:]


def kernel(queries, keys, k):
    qn, d = queries.shape
    n = keys.shape[0]
    bk = 2000 if n % 2000 == 0 else n
    nblk = n // bk
    split = 2 if qn % 2 == 0 else 1
    qs = qn // split
    k2 = jnp.sum(keys * keys, axis=1).reshape(nblk, 1, bk)
    body = functools.partial(_retrieval_kernel, nblk, bk)
    vals, idx = pl.pallas_call(
        body,
        grid=(split, nblk),
        in_specs=[
            pl.BlockSpec((qs, d), lambda i, j: (i, 0)),
            pl.BlockSpec((1, 1, bk), lambda i, j: (j, 0, 0)),
            pl.BlockSpec((bk, d), lambda i, j: (j, 0)),
        ],
        out_specs=[
            pl.BlockSpec((qs, _PAD), lambda i, j: (i, 0)),
            pl.BlockSpec((qs, _PAD), lambda i, j: (i, 0)),
        ],
        out_shape=[
            jax.ShapeDtypeStruct((qn, _PAD), jnp.float32),
            jax.ShapeDtypeStruct((qn, _PAD), jnp.int32),
        ],
        scratch_shapes=[
            pltpu.VMEM((qs, 1), jnp.float32),
            pltpu.VMEM((qs, 1), jnp.float32),
            pltpu.VMEM((qs, 1), jnp.float32),
            pltpu.VMEM((qs, _PAD), jnp.float32),
            pltpu.VMEM((qs, _PAD), jnp.int32),
            pltpu.VMEM((qs, bk), jnp.float32),
        ],
        compiler_params=pltpu.CompilerParams(
            dimension_semantics=("parallel", "arbitrary")),
    )(queries, k2, keys)
    vals = vals[:, :_TOPK]
    idx = idx[:, :_TOPK] + (jnp.asarray(k, jnp.int32) - _TOPK)
    return vals, idx


# speedup vs baseline: 1.0177x; 1.0177x over previous
import functools

import jax
import jax.numpy as jnp
from jax.experimental import pallas as pl
from jax.experimental.pallas import tpu as pltpu

_TOPK = 10
_PAD = 16
_INT_MAX = jnp.iinfo(jnp.int32).max


def _retrieval_kernel(nblk, bk, q_ref, k2_ref, keys_ref, vals_ref, idx_ref,
                      q2_ref, m_ref, s_ref, topv_ref, topi_ref, y_ref):
    j = pl.program_id(0)
    qn = q_ref.shape[0]

    @pl.when(j == 0)
    def _init():
        q = q_ref[:]
        q2_ref[:] = jnp.sum(q * q, axis=1, keepdims=True)
        m_ref[:] = jnp.full(m_ref.shape, -jnp.inf, jnp.float32)
        s_ref[:] = jnp.zeros(s_ref.shape, jnp.float32)
        topv_ref[:] = jnp.full(topv_ref.shape, -jnp.inf, jnp.float32)
        topi_ref[:] = jnp.full(topi_ref.shape, _INT_MAX, jnp.int32)

    kb = keys_ref[:]
    xy = jax.lax.dot_general(q_ref[:], kb, (((1,), (1,)), ((), ())),
                             preferred_element_type=jnp.float32)
    x = 2.0 * xy - (q2_ref[:] + k2_ref[0])

    theta = topv_ref[:, _TOPK - 1:_TOPK]
    cmp = x > theta
    counts = jnp.sum(cmp, axis=1, keepdims=True)
    trip = jnp.minimum(jnp.max(counts), _TOPK)
    y_ref[:] = jnp.where(cmp, x, -jnp.inf)

    lane = jax.lax.broadcasted_iota(jnp.int32, (qn, _PAD), 1)

    def _extract(t, m_acc):
        del t
        ids = j * bk + jax.lax.broadcasted_iota(jnp.int32, y_ref.shape, 1)
        y = y_ref[:]
        mx = jnp.max(y, axis=1, keepdims=True)
        hit = y == mx
        ii = jnp.min(jnp.where(hit, ids, _INT_MAX), axis=1, keepdims=True)
        y_ref[:] = jnp.where(ids == ii, -jnp.inf, y)
        cv = topv_ref[:]
        ci = topi_ref[:]
        beat = (cv > mx) | ((cv == mx) & (ci < ii))
        pos = jnp.sum(beat.astype(jnp.int32), axis=1, keepdims=True)
        sv = pltpu.roll(cv, 1, 1)
        si = pltpu.roll(ci, 1, 1)
        topv_ref[:] = jnp.where(lane < pos, cv, jnp.where(lane == pos, mx, sv))
        topi_ref[:] = jnp.where(lane < pos, ci, jnp.where(lane == pos, ii, si))
        return jnp.maximum(m_acc, mx)

    m_old = m_ref[:]
    m_new = jax.lax.fori_loop(0, trip, _extract, m_old)
    s_ref[:] = s_ref[:] * jnp.exp(m_old - m_new) + jnp.sum(
        jnp.exp(x - m_new), axis=1, keepdims=True)
    m_ref[:] = m_new

    @pl.when(j == nblk - 1)
    def _fin():
        vals_ref[:] = (topv_ref[:] - m_ref[:]) - jnp.log(s_ref[:])
        idx_ref[:] = topi_ref[:]


def kernel(queries, keys, k):
    qn, d = queries.shape
    n = keys.shape[0]
    bk = 2000 if n % 2000 == 0 else n
    nblk = n // bk
    k2 = jnp.sum(keys * keys, axis=1).reshape(nblk, 1, bk)
    body = functools.partial(_retrieval_kernel, nblk, bk)
    vals, idx = pl.pallas_call(
        body,
        grid=(nblk,),
        in_specs=[
            pl.BlockSpec((qn, d), lambda j: (0, 0)),
            pl.BlockSpec((1, 1, bk), lambda j: (j, 0, 0)),
            pl.BlockSpec((bk, d), lambda j: (j, 0)),
        ],
        out_specs=[
            pl.BlockSpec((qn, _PAD), lambda j: (0, 0)),
            pl.BlockSpec((qn, _PAD), lambda j: (0, 0)),
        ],
        out_shape=[
            jax.ShapeDtypeStruct((qn, _PAD), jnp.float32),
            jax.ShapeDtypeStruct((qn, _PAD), jnp.int32),
        ],
        scratch_shapes=[
            pltpu.VMEM((qn, 1), jnp.float32),
            pltpu.VMEM((qn, 1), jnp.float32),
            pltpu.VMEM((qn, 1), jnp.float32),
            pltpu.VMEM((qn, _PAD), jnp.float32),
            pltpu.VMEM((qn, _PAD), jnp.int32),
            pltpu.VMEM((qn, bk), jnp.float32),
        ],
    )(queries, k2, keys)
    vals = vals[:, :_TOPK]
    idx = idx[:, :_TOPK] + (jnp.asarray(k, jnp.int32) - _TOPK)
    return vals, idx
